# baseline (speedup 1.0000x reference)
import jax
import jax.numpy as jnp
from jax import lax
from jax.experimental import pallas as pl
from jax.experimental.pallas import tpu as pltpu

C = 768
LN = 128
SUB = 392
FCHUNK = 56
NSTEPS = SUB // FCHUNK
K = 64
RCHUNK = 128


def _sum_body(x_ref, sums_ref, acc_ref):
    j = pl.program_id(0)

    @pl.when(j == 0)
    def _init():
        acc_ref[...] = jnp.zeros_like(acc_ref)

    acc_ref[...] += jnp.sum(x_ref[...], axis=1)

    @pl.when(j == NSTEPS - 1)
    def _finish():
        sums_ref[...] = jnp.sum(acc_ref[...], axis=1)


def _topk_body(s_ref, idx_ref):
    totals = s_ref[...]
    vj = totals[None, :]
    jj = lax.broadcasted_iota(jnp.int32, (RCHUNK, C), 1)
    ranks = []
    for c in range(C // RCHUNK):
        vi = totals[c * RCHUNK:(c + 1) * RCHUNK][:, None]
        ii = lax.broadcasted_iota(jnp.int32, (RCHUNK, C), 0) + c * RCHUNK
        beats = (vj > vi) | ((vj == vi) & (jj < ii))
        ranks.append(jnp.sum(beats.astype(jnp.int32), axis=1))
    rank = jnp.concatenate(ranks)
    tsel = lax.broadcasted_iota(jnp.int32, (K, C), 0)
    chan = lax.broadcasted_iota(jnp.int32, (K, C), 1)
    onehot = (rank[None, :] == tsel)
    idx_ref[...] = jnp.sum(jnp.where(onehot, chan, 0), axis=1)


def kernel(input):
    x = input.reshape(C, SUB, LN)
    sums = pl.pallas_call(
        _sum_body,
        grid=(NSTEPS,),
        in_specs=[pl.BlockSpec((C, FCHUNK, LN), lambda j: (0, j, 0))],
        out_specs=pl.BlockSpec((C,), lambda j: (0,)),
        out_shape=jax.ShapeDtypeStruct((C,), jnp.float32),
        scratch_shapes=[pltpu.VMEM((C, LN), jnp.float32)],
    )(x)
    return pl.pallas_call(
        _topk_body,
        out_shape=jax.ShapeDtypeStruct((K,), jnp.int32),
    )(sums)
